# Initial kernel scaffold; baseline (speedup 1.0000x reference)
import jax
import jax.numpy as jnp
from jax.experimental import pallas as pl


def kernel(xyz, features, W1, b1, W2, b2, W3, b3):
    raise NotImplementedError("write your pallas kernel here")


# speedup vs baseline: 17.9052x; 17.9052x over previous
import functools

import jax
import jax.numpy as jnp
from jax.experimental import pallas as pl

_RATIO = 0.25
_RADIUS = 0.2
_K = 32
_CH = 128
_SB = 128


def _fps_body(S, xyz_ref, o_ref):
    x = xyz_ref[0]
    y = xyz_ref[1]
    z = xyz_ref[2]
    B, N = x.shape
    iota = jax.lax.broadcasted_iota(jnp.int32, (B, N), 1)
    iota_s = jax.lax.broadcasted_iota(jnp.int32, (B, S), 1)

    def body(i, st):
        dists, far, ax, ay, az = st
        sel = iota == far
        cx = jnp.sum(jnp.where(sel, x, 0.0), axis=1, keepdims=True)
        cy = jnp.sum(jnp.where(sel, y, 0.0), axis=1, keepdims=True)
        cz = jnp.sum(jnp.where(sel, z, 0.0), axis=1, keepdims=True)
        out_sel = iota_s == i
        ax = jnp.where(out_sel, cx, ax)
        ay = jnp.where(out_sel, cy, ay)
        az = jnp.where(out_sel, cz, az)
        dx = x - cx
        dy = y - cy
        dz = z - cz
        d = (dx * dx + dy * dy) + dz * dz
        dists = jnp.minimum(dists, d)
        m = jnp.max(dists, axis=1, keepdims=True)
        far2 = jnp.min(jnp.where(dists == m, iota, N), axis=1, keepdims=True)
        return dists, far2, ax, ay, az

    zs = jnp.zeros((B, S), jnp.float32)
    init = (jnp.full((B, N), 1e10, jnp.float32), jnp.zeros((B, 1), jnp.int32),
            zs, zs, zs)
    _, _, ax, ay, az = jax.lax.fori_loop(0, S, body, init)
    o_ref[0] = ax
    o_ref[1] = ay
    o_ref[2] = az


def _mlp_body(x_ref, w1_ref, b1_ref, w2_ref, b2_ref, w3_ref, b3_ref, o_ref):
    x = x_ref[...]
    h = jnp.maximum(jnp.dot(w1_ref[...], x, preferred_element_type=jnp.float32)
                    + b1_ref[...], 0.0)
    h = jnp.maximum(jnp.dot(w2_ref[...], h, preferred_element_type=jnp.float32)
                    + b2_ref[...], 0.0)
    h = jnp.maximum(jnp.dot(w3_ref[...], h, preferred_element_type=jnp.float32)
                    + b3_ref[...], 0.0)
    o_ref[0] = h.astype(jnp.bfloat16)


def _group_body(xyz_ref, nx_ref, f_ref, o_ref):
    p = xyz_ref[0]
    N = p.shape[0]
    px = p[:, 0:1]
    py = p[:, 1:2]
    pz = p[:, 2:3]
    cx = nx_ref[0, 0:1, :]
    cy = nx_ref[0, 1:2, :]
    cz = nx_ref[0, 2:3, :]
    dx = px - cx
    dy = py - cy
    dz = pz - cz
    d2 = (dx * dx + dy * dy) + dz * dz
    mask = d2 <= jnp.float32(_RADIUS ** 2)
    mb = mask.astype(jnp.bfloat16)
    mf = mask.astype(jnp.float32)

    ri = jax.lax.broadcasted_iota(jnp.int32, (_CH, _CH), 0)
    ci = jax.lax.broadcasted_iota(jnp.int32, (_CH, _CH), 1)
    tri = (ci < ri).astype(jnp.bfloat16)

    sb = mask.shape[1]
    carry = jnp.zeros((1, sb), jnp.float32)
    pos_list = []
    for c in range(N // _CH):
        mb_c = mb[c * _CH:(c + 1) * _CH, :]
        pos_c = jnp.dot(tri, mb_c, preferred_element_type=jnp.float32) + carry
        pos_list.append(pos_c)
        carry = carry + jnp.sum(mf[c * _CH:(c + 1) * _CH, :], axis=0,
                                keepdims=True)
    pos = jnp.concatenate(pos_list, axis=0)

    kf = jnp.float32(_K)
    qsel = jnp.where(mask & (pos < kf), pos, kf + 1.0)

    ft = f_ref[0]
    acc = jnp.zeros((128, sb), jnp.float32)
    for k in range(_K):
        g = (qsel == jnp.float32(k)).astype(jnp.bfloat16)
        acc = jnp.maximum(acc, jnp.dot(ft, g, preferred_element_type=jnp.float32))
    o_ref[0] = acc


def kernel(xyz, features, W1, b1, W2, b2, W3, b3):
    B, N, _ = xyz.shape
    D = features.shape[-1]
    S = int(N * _RATIO)
    BN = B * N

    xyz_c = jnp.transpose(xyz, (2, 0, 1))
    new_xyz_c = pl.pallas_call(
        functools.partial(_fps_body, S),
        out_shape=jax.ShapeDtypeStruct((3, B, S), jnp.float32),
    )(xyz_c)

    feat_t = jnp.transpose(features, (2, 0, 1)).reshape(D, BN)
    cb = 2048
    nchunks = N // cb
    ft3 = pl.pallas_call(
        _mlp_body,
        grid=(B, nchunks),
        in_specs=[
            pl.BlockSpec((D, cb), lambda b, i: (0, b * nchunks + i)),
            pl.BlockSpec((64, D), lambda b, i: (0, 0)),
            pl.BlockSpec((64, 1), lambda b, i: (0, 0)),
            pl.BlockSpec((64, 64), lambda b, i: (0, 0)),
            pl.BlockSpec((64, 1), lambda b, i: (0, 0)),
            pl.BlockSpec((128, 64), lambda b, i: (0, 0)),
            pl.BlockSpec((128, 1), lambda b, i: (0, 0)),
        ],
        out_specs=pl.BlockSpec((1, 128, cb), lambda b, i: (b, 0, i)),
        out_shape=jax.ShapeDtypeStruct((B, 128, N), jnp.bfloat16),
    )(feat_t, W1, b1.reshape(64, 1), W2, b2.reshape(64, 1),
      W3, b3.reshape(128, 1))
    new_xyz_out = jnp.transpose(new_xyz_c, (1, 0, 2))

    new_points = pl.pallas_call(
        _group_body,
        grid=(B, S // _SB),
        in_specs=[
            pl.BlockSpec((1, N, 3), lambda b, s: (b, 0, 0)),
            pl.BlockSpec((1, 3, _SB), lambda b, s: (b, 0, s)),
            pl.BlockSpec((1, 128, N), lambda b, s: (b, 0, 0)),
        ],
        out_specs=pl.BlockSpec((1, 128, _SB), lambda b, s: (b, 0, s)),
        out_shape=jax.ShapeDtypeStruct((B, 128, S), jnp.float32),
    )(xyz, new_xyz_out, ft3)

    return (new_xyz_out, new_points)
